# baseline (speedup 1.0000x reference)
import functools

import jax
import jax.numpy as jnp
from jax import lax
from jax.experimental import pallas as pl
from jax.experimental.pallas import tpu as pltpu
from jax.experimental.pallas import tpu_sc as plsc

N = 10000
E = 320000
D_IN, D_H, D_OUT = 128, 256, 40
DQ = 64
NQ = D_H // DQ
D3P = 64
NC, NS = 2, 16
CHUNK = 128
NP = 10240
NROWSP = 2560
EP = NROWSP * CHUNK
RPSP = NP // NS
NBUF = 5


@functools.cache
def _sc_mesh():
  return plsc.VectorSubcoreMesh(
      core_axis_name="c", subcore_axis_name="s", num_cores=NC, num_subcores=NS)


_SC_PARAMS = pltpu.CompilerParams(use_tc_tiling_on_sc=False)


def _sc_degree(dst_r, ones16):
  rows_pw = NROWSP // (NC * NS)

  @functools.partial(
      pl.kernel,
      out_type=jax.ShapeDtypeStruct((NC * NP, 16), jnp.float32),
      mesh=_sc_mesh(),
      compiler_params=_SC_PARAMS,
      scratch_types=[
          pltpu.VMEM((rows_pw, CHUNK), jnp.int32),
          pltpu.VMEM((CHUNK, 16), jnp.float32),
          pltpu.VMEM_SHARED((NP, 16), jnp.float32),
      ],
  )
  def deg_kernel(dst_hbm, ones_hbm, out_hbm, didx, ones_v, acc):
    cid = lax.axis_index("c")
    sid = lax.axis_index("s")
    wid = cid * NS + sid
    row0 = sid * RPSP
    pltpu.sync_copy(ones_hbm.at[pl.ds(CHUNK, RPSP)], acc.at[pl.ds(row0, RPSP)])
    pltpu.sync_copy(ones_hbm.at[pl.ds(0, CHUNK)], ones_v)
    pltpu.sync_copy(dst_hbm.at[pl.ds(wid * rows_pw, rows_pw)], didx)
    plsc.subcore_barrier()

    def step(j, carry):
      pltpu.sync_copy(ones_v, acc.at[didx.at[j]], add=True)
      return carry

    lax.fori_loop(0, rows_pw, step, 0)
    plsc.subcore_barrier()
    pltpu.sync_copy(acc.at[pl.ds(row0, RPSP)],
                    out_hbm.at[pl.ds(cid * NP + row0, RPSP)])

  return deg_kernel(dst_r, ones16)


def _gather_scatter_loop(tab_hbm, sidx, didx, rows, acc, gsems, ssems,
                         n_chunks):
  for b in range(NBUF):
    pltpu.async_copy(tab_hbm.at[sidx.at[b]], rows.at[b], gsems[b])

  def group(g, carry):
    for b in range(NBUF):
      j = g * NBUF + b
      pltpu.make_async_copy(tab_hbm.at[sidx.at[j]], rows.at[b],
                            gsems[b]).wait()
      pltpu.async_copy(rows.at[b], acc.at[didx.at[j]], ssems[b], add=True)
    for b in range(NBUF):
      j = g * NBUF + b

      @pl.when(j + NBUF < n_chunks)
      def _():
        pltpu.make_async_copy(rows.at[b], acc.at[didx.at[j]],
                              ssems[b]).wait()
        pltpu.async_copy(tab_hbm.at[sidx.at[j + NBUF]], rows.at[b], gsems[b])

    return carry

  assert n_chunks % NBUF == 0
  lax.fori_loop(0, n_chunks // NBUF, group, 0)
  for b in range(NBUF):
    pltpu.make_async_copy(rows.at[b], acc.at[didx.at[n_chunks - NBUF + b]],
                          ssems[b]).wait()


def _sc_aggregate_split(tab, src_r4, dst_r):
  rows_ps = NROWSP // NS

  @functools.partial(
      pl.kernel,
      out_type=jax.ShapeDtypeStruct((4 * NP, DQ), jnp.float32),
      mesh=_sc_mesh(),
      compiler_params=_SC_PARAMS,
      scratch_types=[
          pltpu.VMEM((rows_ps, CHUNK), jnp.int32),
          pltpu.VMEM((rows_ps, CHUNK), jnp.int32),
          pltpu.VMEM((NBUF, CHUNK, DQ), jnp.float32),
          pltpu.VMEM_SHARED((NP, DQ), jnp.float32),
          [pltpu.SemaphoreType.DMA] * NBUF,
          [pltpu.SemaphoreType.DMA] * NBUF,
      ],
  )
  def agg_kernel(tab_hbm, src_hbm, dst_hbm, out_hbm, sidx, didx, rows, acc,
                 gsems, ssems):
    cid = lax.axis_index("c")
    sid = lax.axis_index("s")
    row0 = sid * RPSP
    pltpu.sync_copy(dst_hbm.at[pl.ds(sid * rows_ps, rows_ps)], didx)
    for p in range(2):
      q = cid * 2 + p
      pltpu.sync_copy(tab_hbm.at[pl.ds(q * NP + row0, RPSP)],
                      acc.at[pl.ds(row0, RPSP)])
      pltpu.sync_copy(
          src_hbm.at[pl.ds(q * NROWSP + sid * rows_ps, rows_ps)], sidx)
      plsc.subcore_barrier()
      _gather_scatter_loop(tab_hbm, sidx, didx, rows, acc, gsems, ssems, rows_ps)
      plsc.subcore_barrier()
      pltpu.sync_copy(acc.at[pl.ds(row0, RPSP)],
                      out_hbm.at[pl.ds(q * NP + row0, RPSP)])
      if p == 0:
        plsc.subcore_barrier()

    return

  return agg_kernel(tab, src_r4, dst_r)


def _sc_aggregate_edgesplit(tab, zeros64, src_r, dst_r):
  rows_pw = NROWSP // (NC * NS)

  @functools.partial(
      pl.kernel,
      out_type=jax.ShapeDtypeStruct((NC * NP, D3P), jnp.float32),
      mesh=_sc_mesh(),
      compiler_params=_SC_PARAMS,
      scratch_types=[
          pltpu.VMEM((rows_pw, CHUNK), jnp.int32),
          pltpu.VMEM((rows_pw, CHUNK), jnp.int32),
          pltpu.VMEM((NBUF, CHUNK, D3P), jnp.float32),
          pltpu.VMEM_SHARED((NP, D3P), jnp.float32),
          [pltpu.SemaphoreType.DMA] * NBUF,
          [pltpu.SemaphoreType.DMA] * NBUF,
      ],
  )
  def agg3_kernel(tab_hbm, zero_hbm, src_hbm, dst_hbm, out_hbm, sidx, didx,
                  rows, acc, gsems, ssems):
    cid = lax.axis_index("c")
    sid = lax.axis_index("s")
    wid = cid * NS + sid
    row0 = sid * RPSP

    @pl.when(cid == 0)
    def _():
      pltpu.sync_copy(tab_hbm.at[pl.ds(row0, RPSP)], acc.at[pl.ds(row0, RPSP)])

    @pl.when(cid == 1)
    def _():
      pltpu.sync_copy(zero_hbm.at[pl.ds(row0, RPSP)],
                      acc.at[pl.ds(row0, RPSP)])

    pltpu.sync_copy(src_hbm.at[pl.ds(wid * rows_pw, rows_pw)], sidx)
    pltpu.sync_copy(dst_hbm.at[pl.ds(wid * rows_pw, rows_pw)], didx)
    plsc.subcore_barrier()
    _gather_scatter_loop(tab_hbm, sidx, didx, rows, acc, gsems, ssems, rows_pw)
    plsc.subcore_barrier()
    pltpu.sync_copy(acc.at[pl.ds(row0, RPSP)],
                    out_hbm.at[pl.ds(cid * NP + row0, RPSP)])

  return agg3_kernel(tab, zeros64, src_r, dst_r)


def _dinv_from_degp(degp):
  cnt = degp[0:N, 0:1] + degp[NP:NP + N, 0:1] + 1.0
  return lax.rsqrt(cnt)


def _write_quarters(tab_ref, hs):
  pad = jnp.zeros((NP - N, DQ), jnp.float32)
  for q in range(NQ):
    tab_ref[q * NP:q * NP + N, :] = hs[:, q * DQ:(q + 1) * DQ]
    tab_ref[q * NP + N:(q + 1) * NP, :] = pad


def _read_quarters(agg):
  return jnp.concatenate([agg[q * NP:q * NP + N, :] for q in range(NQ)],
                         axis=1)


def _tc_pre_body(x_ref, w1_ref, degp_ref, tab_ref):
  dinv = _dinv_from_degp(degp_ref[...])
  h = jnp.dot(x_ref[...], w1_ref[...], preferred_element_type=jnp.float32)
  _write_quarters(tab_ref, h * dinv)


def _tc_bnrelu_body(agg_ref, degp_ref, b_ref, g_ref, bt_ref, h_ref):
  dinv = _dinv_from_degp(degp_ref[...])
  z = _read_quarters(agg_ref[...])
  z = z * dinv + b_ref[...]
  mu = jnp.mean(z, axis=0, keepdims=True)
  var = jnp.mean((z - mu) ** 2, axis=0, keepdims=True)
  h = (z - mu) * lax.rsqrt(var + 1e-5) * g_ref[...] + bt_ref[...]
  h_ref[...] = jnp.maximum(h, 0.0)


def _tc_matmul_body(split_out, h_ref, degp_ref, w_ref, tab_ref):
  dinv = _dinv_from_degp(degp_ref[...])
  hn = jnp.dot(h_ref[...], w_ref[...], preferred_element_type=jnp.float32)
  hn = hn * dinv
  if split_out:
    _write_quarters(tab_ref, hn)
  else:
    tab_ref[0:N, 0:D_OUT] = hn
    tab_ref[0:N, D_OUT:D3P] = jnp.zeros((N, D3P - D_OUT), jnp.float32)
    tab_ref[N:NP, :] = jnp.zeros((NP - N, D3P), jnp.float32)


def _tc_final_body(agg3_ref, degp_ref, b3_ref, out_ref):
  dinv = _dinv_from_degp(degp_ref[...])
  z = (agg3_ref[0:N, 0:D_OUT] + agg3_ref[NP:NP + N, 0:D_OUT]) * dinv
  z = z + b3_ref[...]
  m = jnp.max(z, axis=1, keepdims=True)
  lse = jnp.log(jnp.sum(jnp.exp(z - m), axis=1, keepdims=True)) + m
  out_ref[...] = z - lse


def _tc_pre(x, W1, degp):
  return pl.pallas_call(
      _tc_pre_body,
      out_shape=jax.ShapeDtypeStruct((NQ * NP, DQ), jnp.float32),
  )(x, W1, degp)


def _tc_mid(agg, degp, b, gamma, beta, W, split_out):
  h = pl.pallas_call(
      _tc_bnrelu_body,
      out_shape=jax.ShapeDtypeStruct((N, D_H), jnp.float32),
  )(agg, degp, b, gamma, beta)
  out_shape = (jax.ShapeDtypeStruct((NQ * NP, DQ), jnp.float32) if split_out
               else jax.ShapeDtypeStruct((NP, D3P), jnp.float32))
  return pl.pallas_call(
      functools.partial(_tc_matmul_body, split_out),
      out_shape=out_shape,
  )(h, degp, W)


def _tc_final(agg3, degp, b3):
  return pl.pallas_call(
      _tc_final_body,
      out_shape=jax.ShapeDtypeStruct((N, D_OUT), jnp.float32),
  )(agg3, degp, b3)


def kernel(x, edge_index, W1, b1, W2, b2, W3, b3, gamma, beta):
  src = edge_index[0]
  dst = edge_index[1]
  pad_e = EP - E
  src_p = jnp.concatenate([src, jnp.zeros((pad_e,), src.dtype)])
  trash = N + (jnp.arange(pad_e, dtype=dst.dtype) % 8)
  dst_p = jnp.concatenate([dst, trash])
  src_r = src_p.reshape(NROWSP, CHUNK)
  dst_r = dst_p.reshape(NROWSP, CHUNK)
  src_r4 = jnp.concatenate([src_r + q * NP for q in range(NQ)], axis=0)
  ones16 = jnp.concatenate(
      [jnp.ones((CHUNK, 16), jnp.float32),
       jnp.zeros((RPSP, 16), jnp.float32)], axis=0)
  zeros64 = jnp.zeros((NP, D3P), jnp.float32)
  b1r = b1.reshape(1, D_H)
  b2r = b2.reshape(1, D_H)
  b3r = b3.reshape(1, D_OUT)
  gr = gamma.reshape(1, D_H)
  btr = beta.reshape(1, D_H)

  degp = _sc_degree(dst_r, ones16)
  tab1 = _tc_pre(x, W1, degp)
  agg1 = _sc_aggregate_split(tab1, src_r4, dst_r)
  tab2 = _tc_mid(agg1, degp, b1r, gr, btr, W2, split_out=True)
  agg2 = _sc_aggregate_split(tab2, src_r4, dst_r)
  tab3 = _tc_mid(agg2, degp, b2r, gr, btr, W3, split_out=False)
  agg3 = _sc_aggregate_edgesplit(tab3, zeros64, src_r, dst_r)
  return _tc_final(agg3, degp, b3r)


# speedup vs baseline: 1.7645x; 1.7645x over previous
import functools

import jax
import jax.numpy as jnp
from jax import lax
from jax.experimental import pallas as pl
from jax.experimental.pallas import tpu as pltpu
from jax.experimental.pallas import tpu_sc as plsc

N = 10000
E = 320000
D_IN, D_H, D_OUT = 128, 256, 40
DQ = 32
NQ = D_H // DQ
NQ3 = 2
D3P = NQ3 * DQ
NC, NS = 2, 16
CHUNK = 128
NP = 10240
NROWSP = 2560
EP = NROWSP * CHUNK
RPSP = NP // NS
NBUF = 5


@functools.cache
def _sc_mesh():
  return plsc.VectorSubcoreMesh(
      core_axis_name="c", subcore_axis_name="s", num_cores=NC, num_subcores=NS)


_SC_PARAMS = pltpu.CompilerParams(use_tc_tiling_on_sc=False)


def _sc_degree(dst_r, ones16):
  rows_pw = NROWSP // (NC * NS)

  @functools.partial(
      pl.kernel,
      out_type=jax.ShapeDtypeStruct((NC * NP, 16), jnp.float32),
      mesh=_sc_mesh(),
      compiler_params=_SC_PARAMS,
      scratch_types=[
          pltpu.VMEM((rows_pw, CHUNK), jnp.int32),
          pltpu.VMEM((CHUNK, 16), jnp.float32),
          pltpu.VMEM_SHARED((NP, 16), jnp.float32),
      ],
  )
  def deg_kernel(dst_hbm, ones_hbm, out_hbm, didx, ones_v, acc):
    cid = lax.axis_index("c")
    sid = lax.axis_index("s")
    wid = cid * NS + sid
    row0 = sid * RPSP
    pltpu.sync_copy(ones_hbm.at[pl.ds(CHUNK, RPSP)], acc.at[pl.ds(row0, RPSP)])
    pltpu.sync_copy(ones_hbm.at[pl.ds(0, CHUNK)], ones_v)
    pltpu.sync_copy(dst_hbm.at[pl.ds(wid * rows_pw, rows_pw)], didx)
    plsc.subcore_barrier()

    def step(j, carry):
      pltpu.sync_copy(ones_v, acc.at[didx.at[j]], add=True)
      return carry

    lax.fori_loop(0, rows_pw, step, 0)
    plsc.subcore_barrier()
    pltpu.sync_copy(acc.at[pl.ds(row0, RPSP)],
                    out_hbm.at[pl.ds(cid * NP + row0, RPSP)])

  return deg_kernel(dst_r, ones16)


def _gather_scatter_loop(tab_hbm, sidx, didx, rows, acc, gsems, ssems,
                         n_chunks):
  for b in range(NBUF):
    pltpu.async_copy(tab_hbm.at[sidx.at[b]], rows.at[b], gsems[b])

  def group(g, carry):
    for b in range(NBUF):
      j = g * NBUF + b
      pltpu.make_async_copy(tab_hbm.at[sidx.at[j]], rows.at[b],
                            gsems[b]).wait()
      pltpu.async_copy(rows.at[b], acc.at[didx.at[j]], ssems[b], add=True)
    for b in range(NBUF):
      j = g * NBUF + b

      @pl.when(j + NBUF < n_chunks)
      def _():
        pltpu.make_async_copy(rows.at[b], acc.at[didx.at[j]],
                              ssems[b]).wait()
        pltpu.async_copy(tab_hbm.at[sidx.at[j + NBUF]], rows.at[b], gsems[b])

    return carry

  assert n_chunks % NBUF == 0
  lax.fori_loop(0, n_chunks // NBUF, group, 0)
  for b in range(NBUF):
    pltpu.make_async_copy(rows.at[b], acc.at[didx.at[n_chunks - NBUF + b]],
                          ssems[b]).wait()


def _sc_aggregate(tab, src_r, dst_r, nq):
  rows_ps = NROWSP // NS
  ppc = nq // NC

  @functools.partial(
      pl.kernel,
      out_type=jax.ShapeDtypeStruct((NP, nq * DQ), jnp.float32),
      mesh=_sc_mesh(),
      compiler_params=_SC_PARAMS,
      scratch_types=[
          pltpu.VMEM((rows_ps, CHUNK), jnp.int32),
          pltpu.VMEM((rows_ps, CHUNK), jnp.int32),
          pltpu.VMEM((NBUF, CHUNK, DQ), jnp.float32),
          pltpu.VMEM_SHARED((NP, DQ), jnp.float32),
          pltpu.VMEM_SHARED((NP, DQ), jnp.float32),
          [pltpu.SemaphoreType.DMA] * NBUF,
          [pltpu.SemaphoreType.DMA] * NBUF,
      ],
  )
  def agg_kernel(tab_hbm, src_hbm, dst_hbm, out_hbm, sidx, didx, rows, tspm,
                 acc, gsems, ssems):
    cid = lax.axis_index("c")
    sid = lax.axis_index("s")
    row0 = sid * RPSP
    pltpu.sync_copy(src_hbm.at[pl.ds(sid * rows_ps, rows_ps)], sidx)
    pltpu.sync_copy(dst_hbm.at[pl.ds(sid * rows_ps, rows_ps)], didx)
    for p in range(ppc):
      q = cid * ppc + p
      pltpu.sync_copy(tab_hbm.at[pl.ds(row0, RPSP), pl.ds(q * DQ, DQ)],
                      tspm.at[pl.ds(row0, RPSP)])
      pltpu.sync_copy(tab_hbm.at[pl.ds(row0, RPSP), pl.ds(q * DQ, DQ)],
                      acc.at[pl.ds(row0, RPSP)])
      plsc.subcore_barrier()
      _gather_scatter_loop(tspm, sidx, didx, rows, acc, gsems, ssems,
                           rows_ps)
      plsc.subcore_barrier()
      pltpu.sync_copy(acc.at[pl.ds(row0, RPSP)],
                      out_hbm.at[pl.ds(row0, RPSP), pl.ds(q * DQ, DQ)])

  return agg_kernel(tab, src_r, dst_r)


def _dinv_from_degp(degp):
  cnt = degp[0:N, 0:1] + degp[NP:NP + N, 0:1] + 1.0
  return lax.rsqrt(cnt)


def _tc_pre_body(x_ref, w1_ref, degp_ref, tab_ref):
  dinv = _dinv_from_degp(degp_ref[...])
  h = jnp.dot(x_ref[...], w1_ref[...], preferred_element_type=jnp.float32)
  tab_ref[0:N, :] = h * dinv
  tab_ref[N:NP, :] = jnp.zeros((NP - N, D_H), jnp.float32)


def _tc_bnrelu_body(agg_ref, degp_ref, b_ref, g_ref, bt_ref, h_ref):
  dinv = _dinv_from_degp(degp_ref[...])
  z = agg_ref[0:N, :] * dinv + b_ref[...]
  mu = jnp.mean(z, axis=0, keepdims=True)
  var = jnp.mean((z - mu) ** 2, axis=0, keepdims=True)
  h = (z - mu) * lax.rsqrt(var + 1e-5) * g_ref[...] + bt_ref[...]
  h_ref[...] = jnp.maximum(h, 0.0)


def _tc_matmul_body(split_out, h_ref, degp_ref, w_ref, tab_ref):
  dinv = _dinv_from_degp(degp_ref[...])
  hn = jnp.dot(h_ref[...], w_ref[...], preferred_element_type=jnp.float32)
  hn = hn * dinv
  if split_out:
    tab_ref[0:N, :] = hn
    tab_ref[N:NP, :] = jnp.zeros((NP - N, D_H), jnp.float32)
  else:
    tab_ref[0:N, 0:D_OUT] = hn
    tab_ref[0:N, D_OUT:D3P] = jnp.zeros((N, D3P - D_OUT), jnp.float32)
    tab_ref[N:NP, :] = jnp.zeros((NP - N, D3P), jnp.float32)


def _tc_final_body(agg3_ref, degp_ref, b3_ref, out_ref):
  dinv = _dinv_from_degp(degp_ref[...])
  z = agg3_ref[0:N, 0:D_OUT] * dinv + b3_ref[...]
  m = jnp.max(z, axis=1, keepdims=True)
  lse = jnp.log(jnp.sum(jnp.exp(z - m), axis=1, keepdims=True)) + m
  out_ref[...] = z - lse


def _tc_pre(x, W1, degp):
  return pl.pallas_call(
      _tc_pre_body,
      out_shape=jax.ShapeDtypeStruct((NP, D_H), jnp.float32),
  )(x, W1, degp)


def _tc_mid(agg, degp, b, gamma, beta, W, split_out):
  h = pl.pallas_call(
      _tc_bnrelu_body,
      out_shape=jax.ShapeDtypeStruct((N, D_H), jnp.float32),
  )(agg, degp, b, gamma, beta)
  out_shape = jax.ShapeDtypeStruct(
      (NP, D_H if split_out else D3P), jnp.float32)
  return pl.pallas_call(
      functools.partial(_tc_matmul_body, split_out),
      out_shape=out_shape,
  )(h, degp, W)


def _tc_final(agg3, degp, b3):
  return pl.pallas_call(
      _tc_final_body,
      out_shape=jax.ShapeDtypeStruct((N, D_OUT), jnp.float32),
  )(agg3, degp, b3)


def kernel(x, edge_index, W1, b1, W2, b2, W3, b3, gamma, beta):
  src = edge_index[0]
  dst = edge_index[1]
  pad_e = EP - E
  src_p = jnp.concatenate([src, jnp.zeros((pad_e,), src.dtype)])
  trash = N + (jnp.arange(pad_e, dtype=dst.dtype) % 8)
  dst_p = jnp.concatenate([dst, trash])
  src_r = src_p.reshape(NROWSP, CHUNK)
  dst_r = dst_p.reshape(NROWSP, CHUNK)
  ones16 = jnp.concatenate(
      [jnp.ones((CHUNK, 16), jnp.float32),
       jnp.zeros((RPSP, 16), jnp.float32)], axis=0)
  b1r = b1.reshape(1, D_H)
  b2r = b2.reshape(1, D_H)
  b3r = b3.reshape(1, D_OUT)
  gr = gamma.reshape(1, D_H)
  btr = beta.reshape(1, D_H)

  degp = _sc_degree(dst_r, ones16)
  tab1 = _tc_pre(x, W1, degp)
  agg1 = _sc_aggregate(tab1, src_r, dst_r, NQ)
  tab2 = _tc_mid(agg1, degp, b1r, gr, btr, W2, split_out=True)
  agg2 = _sc_aggregate(tab2, src_r, dst_r, NQ)
  tab3 = _tc_mid(agg2, degp, b2r, gr, btr, W3, split_out=False)
  agg3 = _sc_aggregate(tab3, src_r, dst_r, NQ3)
  return _tc_final(agg3, degp, b3r)


# speedup vs baseline: 1.8340x; 1.0394x over previous
import functools

import jax
import jax.numpy as jnp
from jax import lax
from jax.experimental import pallas as pl
from jax.experimental.pallas import tpu as pltpu
from jax.experimental.pallas import tpu_sc as plsc

N = 10000
E = 320000
D_IN, D_H, D_OUT = 128, 256, 40
DQ = 32
NQ = D_H // DQ
NQ3 = 2
D3P = NQ3 * DQ
NC, NS = 2, 16
CHUNK = 128
NP = 10240
NROWSP = 2560
EP = NROWSP * CHUNK
RPSP = NP // NS
NBUF = 8


@functools.cache
def _sc_mesh():
  return plsc.VectorSubcoreMesh(
      core_axis_name="c", subcore_axis_name="s", num_cores=NC, num_subcores=NS)


_SC_PARAMS = pltpu.CompilerParams(use_tc_tiling_on_sc=False)


def _sc_degree(dst_r, ones16):
  rows_pw = NROWSP // (NC * NS)

  @functools.partial(
      pl.kernel,
      out_type=jax.ShapeDtypeStruct((NC * NP, 16), jnp.float32),
      mesh=_sc_mesh(),
      compiler_params=_SC_PARAMS,
      scratch_types=[
          pltpu.VMEM((rows_pw, CHUNK), jnp.int32),
          pltpu.VMEM((CHUNK, 16), jnp.float32),
          pltpu.VMEM_SHARED((NP, 16), jnp.float32),
      ],
  )
  def deg_kernel(dst_hbm, ones_hbm, out_hbm, didx, ones_v, acc):
    cid = lax.axis_index("c")
    sid = lax.axis_index("s")
    wid = cid * NS + sid
    row0 = sid * RPSP
    pltpu.sync_copy(ones_hbm.at[pl.ds(CHUNK, RPSP)], acc.at[pl.ds(row0, RPSP)])
    pltpu.sync_copy(ones_hbm.at[pl.ds(0, CHUNK)], ones_v)
    pltpu.sync_copy(dst_hbm.at[pl.ds(wid * rows_pw, rows_pw)], didx)
    plsc.subcore_barrier()

    def step(j, carry):
      pltpu.sync_copy(ones_v, acc.at[didx.at[j]], add=True)
      return carry

    lax.fori_loop(0, rows_pw, step, 0)
    plsc.subcore_barrier()
    pltpu.sync_copy(acc.at[pl.ds(row0, RPSP)],
                    out_hbm.at[pl.ds(cid * NP + row0, RPSP)])

  return deg_kernel(dst_r, ones16)


def _gather_scatter_loop(tab_hbm, sidx, didx, rows, acc, gsems, ssems,
                         n_chunks):
  for b in range(NBUF):
    pltpu.async_copy(tab_hbm.at[sidx.at[b]], rows.at[b], gsems[b])

  def group(g, carry):
    for b in range(NBUF):
      j = g * NBUF + b
      pltpu.make_async_copy(tab_hbm.at[sidx.at[j]], rows.at[b],
                            gsems[b]).wait()
      pltpu.async_copy(rows.at[b], acc.at[didx.at[j]], ssems[b], add=True)
    for b in range(NBUF):
      j = g * NBUF + b

      @pl.when(j + NBUF < n_chunks)
      def _():
        pltpu.make_async_copy(rows.at[b], acc.at[didx.at[j]],
                              ssems[b]).wait()
        pltpu.async_copy(tab_hbm.at[sidx.at[j + NBUF]], rows.at[b], gsems[b])

    return carry

  assert n_chunks % NBUF == 0
  lax.fori_loop(0, n_chunks // NBUF, group, 0)
  for b in range(NBUF):
    pltpu.make_async_copy(rows.at[b], acc.at[didx.at[n_chunks - NBUF + b]],
                          ssems[b]).wait()


def _sc_aggregate(tab, src_r, dst_r, nq):
  rows_ps = NROWSP // NS
  ppc = nq // NC

  @functools.partial(
      pl.kernel,
      out_type=jax.ShapeDtypeStruct((NP, nq * DQ), jnp.float32),
      mesh=_sc_mesh(),
      compiler_params=_SC_PARAMS,
      scratch_types=[
          pltpu.VMEM((rows_ps, CHUNK), jnp.int32),
          pltpu.VMEM((rows_ps, CHUNK), jnp.int32),
          pltpu.VMEM((NBUF, CHUNK, DQ), jnp.float32),
          pltpu.VMEM_SHARED((NP, DQ), jnp.float32),
          pltpu.VMEM_SHARED((NP, DQ), jnp.float32),
          [pltpu.SemaphoreType.DMA] * NBUF,
          [pltpu.SemaphoreType.DMA] * NBUF,
      ],
  )
  def agg_kernel(tab_hbm, src_hbm, dst_hbm, out_hbm, sidx, didx, rows, tspm,
                 acc, gsems, ssems):
    cid = lax.axis_index("c")
    sid = lax.axis_index("s")
    row0 = sid * RPSP
    pltpu.sync_copy(src_hbm.at[pl.ds(sid * rows_ps, rows_ps)], sidx)
    pltpu.sync_copy(dst_hbm.at[pl.ds(sid * rows_ps, rows_ps)], didx)
    for p in range(ppc):
      q = cid * ppc + p
      pltpu.sync_copy(tab_hbm.at[pl.ds(row0, RPSP), pl.ds(q * DQ, DQ)],
                      tspm.at[pl.ds(row0, RPSP)])
      pltpu.sync_copy(tab_hbm.at[pl.ds(row0, RPSP), pl.ds(q * DQ, DQ)],
                      acc.at[pl.ds(row0, RPSP)])
      plsc.subcore_barrier()
      _gather_scatter_loop(tspm, sidx, didx, rows, acc, gsems, ssems,
                           rows_ps)
      plsc.subcore_barrier()
      pltpu.sync_copy(acc.at[pl.ds(row0, RPSP)],
                      out_hbm.at[pl.ds(row0, RPSP), pl.ds(q * DQ, DQ)])

  return agg_kernel(tab, src_r, dst_r)


def _dinv_from_degp(degp):
  cnt = degp[0:N, 0:1] + degp[NP:NP + N, 0:1] + 1.0
  return lax.rsqrt(cnt)


def _tc_pre_body(x_ref, w1_ref, degp_ref, tab_ref):
  dinv = _dinv_from_degp(degp_ref[...])
  h = jnp.dot(x_ref[...], w1_ref[...], preferred_element_type=jnp.float32)
  tab_ref[0:N, :] = h * dinv
  tab_ref[N:NP, :] = jnp.zeros((NP - N, D_H), jnp.float32)


def _tc_bnrelu_body(agg_ref, degp_ref, b_ref, g_ref, bt_ref, h_ref):
  dinv = _dinv_from_degp(degp_ref[...])
  z = agg_ref[0:N, :] * dinv + b_ref[...]
  mu = jnp.mean(z, axis=0, keepdims=True)
  var = jnp.mean((z - mu) ** 2, axis=0, keepdims=True)
  h = (z - mu) * lax.rsqrt(var + 1e-5) * g_ref[...] + bt_ref[...]
  h_ref[...] = jnp.maximum(h, 0.0)


def _tc_matmul_body(split_out, h_ref, degp_ref, w_ref, tab_ref):
  dinv = _dinv_from_degp(degp_ref[...])
  hn = jnp.dot(h_ref[...], w_ref[...], preferred_element_type=jnp.float32)
  hn = hn * dinv
  if split_out:
    tab_ref[0:N, :] = hn
    tab_ref[N:NP, :] = jnp.zeros((NP - N, D_H), jnp.float32)
  else:
    tab_ref[0:N, 0:D_OUT] = hn
    tab_ref[0:N, D_OUT:D3P] = jnp.zeros((N, D3P - D_OUT), jnp.float32)
    tab_ref[N:NP, :] = jnp.zeros((NP - N, D3P), jnp.float32)


def _tc_final_body(agg3_ref, degp_ref, b3_ref, out_ref):
  dinv = _dinv_from_degp(degp_ref[...])
  z = agg3_ref[0:N, 0:D_OUT] * dinv + b3_ref[...]
  m = jnp.max(z, axis=1, keepdims=True)
  lse = jnp.log(jnp.sum(jnp.exp(z - m), axis=1, keepdims=True)) + m
  out_ref[...] = z - lse


def _tc_pre(x, W1, degp):
  return pl.pallas_call(
      _tc_pre_body,
      out_shape=jax.ShapeDtypeStruct((NP, D_H), jnp.float32),
  )(x, W1, degp)


def _tc_mid(agg, degp, b, gamma, beta, W, split_out):
  h = pl.pallas_call(
      _tc_bnrelu_body,
      out_shape=jax.ShapeDtypeStruct((N, D_H), jnp.float32),
  )(agg, degp, b, gamma, beta)
  out_shape = jax.ShapeDtypeStruct(
      (NP, D_H if split_out else D3P), jnp.float32)
  return pl.pallas_call(
      functools.partial(_tc_matmul_body, split_out),
      out_shape=out_shape,
  )(h, degp, W)


def _tc_final(agg3, degp, b3):
  return pl.pallas_call(
      _tc_final_body,
      out_shape=jax.ShapeDtypeStruct((N, D_OUT), jnp.float32),
  )(agg3, degp, b3)


def kernel(x, edge_index, W1, b1, W2, b2, W3, b3, gamma, beta):
  src = edge_index[0]
  dst = edge_index[1]
  pad_e = EP - E
  src_p = jnp.concatenate([src, jnp.zeros((pad_e,), src.dtype)])
  trash = N + (jnp.arange(pad_e, dtype=dst.dtype) % 8)
  dst_p = jnp.concatenate([dst, trash])
  src_r = src_p.reshape(NROWSP, CHUNK)
  dst_r = dst_p.reshape(NROWSP, CHUNK)
  ones16 = jnp.concatenate(
      [jnp.ones((CHUNK, 16), jnp.float32),
       jnp.zeros((RPSP, 16), jnp.float32)], axis=0)
  b1r = b1.reshape(1, D_H)
  b2r = b2.reshape(1, D_H)
  b3r = b3.reshape(1, D_OUT)
  gr = gamma.reshape(1, D_H)
  btr = beta.reshape(1, D_H)

  degp = _sc_degree(dst_r, ones16)
  tab1 = _tc_pre(x, W1, degp)
  agg1 = _sc_aggregate(tab1, src_r, dst_r, NQ)
  tab2 = _tc_mid(agg1, degp, b1r, gr, btr, W2, split_out=True)
  agg2 = _sc_aggregate(tab2, src_r, dst_r, NQ)
  tab3 = _tc_mid(agg2, degp, b2r, gr, btr, W3, split_out=False)
  agg3 = _sc_aggregate(tab3, src_r, dst_r, NQ3)
  return _tc_final(agg3, degp, b3r)


# speedup vs baseline: 1.8368x; 1.0016x over previous
import functools

import jax
import jax.numpy as jnp
from jax import lax
from jax.experimental import pallas as pl
from jax.experimental.pallas import tpu as pltpu
from jax.experimental.pallas import tpu_sc as plsc

N = 10000
E = 320000
D_IN, D_H, D_OUT = 128, 256, 40
DQ = 32
NQ = D_H // DQ
NQ3 = 2
D3P = NQ3 * DQ
NC, NS = 2, 16
CHUNK = 128
NP = 10240
NROWSP = 2560
EP = NROWSP * CHUNK
RPSP = NP // NS
NBUF = 8


@functools.cache
def _sc_mesh():
  return plsc.VectorSubcoreMesh(
      core_axis_name="c", subcore_axis_name="s", num_cores=NC, num_subcores=NS)


_SC_PARAMS = pltpu.CompilerParams(use_tc_tiling_on_sc=False)


def _sc_degree(dst_r, ones16):
  rows_pw = NROWSP // (NC * NS)

  @functools.partial(
      pl.kernel,
      out_type=jax.ShapeDtypeStruct((NC * NP, 16), jnp.float32),
      mesh=_sc_mesh(),
      compiler_params=_SC_PARAMS,
      scratch_types=[
          pltpu.VMEM((rows_pw, CHUNK), jnp.int32),
          pltpu.VMEM((CHUNK, 16), jnp.float32),
          pltpu.VMEM_SHARED((NP, 16), jnp.float32),
      ],
  )
  def deg_kernel(dst_hbm, ones_hbm, out_hbm, didx, ones_v, acc):
    cid = lax.axis_index("c")
    sid = lax.axis_index("s")
    wid = cid * NS + sid
    row0 = sid * RPSP
    pltpu.sync_copy(ones_hbm.at[pl.ds(CHUNK, RPSP)], acc.at[pl.ds(row0, RPSP)])
    pltpu.sync_copy(ones_hbm.at[pl.ds(0, CHUNK)], ones_v)
    pltpu.sync_copy(dst_hbm.at[pl.ds(wid * rows_pw, rows_pw)], didx)
    plsc.subcore_barrier()

    def step(j, carry):
      pltpu.sync_copy(ones_v, acc.at[didx.at[j]], add=True)
      return carry

    lax.fori_loop(0, rows_pw, step, 0)
    plsc.subcore_barrier()
    pltpu.sync_copy(acc.at[pl.ds(row0, RPSP)],
                    out_hbm.at[pl.ds(cid * NP + row0, RPSP)])

  return deg_kernel(dst_r, ones16)


def _gather_scatter_loop(tab_ref, sidx, didx, rows, acc, gsems, ssems,
                         n_chunks):
  for b in range(NBUF):
    pltpu.async_copy(tab_ref.at[sidx.at[b]], rows.at[b], gsems[b])

  def group(g, carry):
    for b in range(NBUF):
      j = g * NBUF + b
      pltpu.make_async_copy(tab_ref.at[sidx.at[j]], rows.at[b],
                            gsems[b]).wait()
      pltpu.async_copy(rows.at[b], acc.at[didx.at[j]], ssems[b], add=True)
    for b in range(NBUF):
      j = g * NBUF + b

      @pl.when(j + NBUF < n_chunks)
      def _():
        pltpu.make_async_copy(rows.at[b], acc.at[didx.at[j]],
                              ssems[b]).wait()
        pltpu.async_copy(tab_ref.at[sidx.at[j + NBUF]], rows.at[b], gsems[b])

    return carry

  assert n_chunks % NBUF == 0
  lax.fori_loop(0, n_chunks // NBUF, group, 0)
  for b in range(NBUF):
    pltpu.make_async_copy(rows.at[b], acc.at[didx.at[n_chunks - NBUF + b]],
                          ssems[b]).wait()


def _sc_aggregate(tab, src_r, dst_r, nq):
  rows_ps = NROWSP // NS
  ppc = nq // NC

  @functools.partial(
      pl.kernel,
      out_type=jax.ShapeDtypeStruct((NP, nq * DQ), jnp.float32),
      mesh=_sc_mesh(),
      compiler_params=_SC_PARAMS,
      scratch_types=[
          pltpu.VMEM((rows_ps, CHUNK), jnp.int32),
          pltpu.VMEM((rows_ps, CHUNK), jnp.int32),
          pltpu.VMEM((NBUF, CHUNK, DQ), jnp.float32),
          pltpu.VMEM_SHARED((NP, DQ), jnp.float32),
          pltpu.VMEM_SHARED((NP, DQ), jnp.float32),
          [pltpu.SemaphoreType.DMA] * NBUF,
          [pltpu.SemaphoreType.DMA] * NBUF,
      ],
  )
  def agg_kernel(tab_hbm, src_hbm, dst_hbm, out_hbm, sidx, didx, rows, tspm,
                 acc, gsems, ssems):
    cid = lax.axis_index("c")
    sid = lax.axis_index("s")
    row0 = sid * RPSP
    pltpu.sync_copy(src_hbm.at[pl.ds(sid * rows_ps, rows_ps)], sidx)
    pltpu.sync_copy(dst_hbm.at[pl.ds(sid * rows_ps, rows_ps)], didx)
    for p in range(ppc):
      q = cid * ppc + p
      pltpu.sync_copy(tab_hbm.at[pl.ds(row0, RPSP), pl.ds(q * DQ, DQ)],
                      tspm.at[pl.ds(row0, RPSP)])
      pltpu.sync_copy(tab_hbm.at[pl.ds(row0, RPSP), pl.ds(q * DQ, DQ)],
                      acc.at[pl.ds(row0, RPSP)])
      plsc.subcore_barrier()
      _gather_scatter_loop(tspm, sidx, didx, rows, acc, gsems, ssems,
                           rows_ps)
      plsc.subcore_barrier()
      pltpu.sync_copy(acc.at[pl.ds(row0, RPSP)],
                      out_hbm.at[pl.ds(row0, RPSP), pl.ds(q * DQ, DQ)])

  return agg_kernel(tab, src_r, dst_r)


def _dinv_from_degp(degp):
  cnt = degp[0:N, 0:1] + degp[NP:NP + N, 0:1] + 1.0
  return lax.rsqrt(cnt)


def _tc_pre_body(x_ref, w1_ref, degp_ref, tab_ref):
  dinv = _dinv_from_degp(degp_ref[...])
  h = jnp.dot(x_ref[...], w1_ref[...], preferred_element_type=jnp.float32)
  tab_ref[0:N, :] = h * dinv
  tab_ref[N:NP, :] = jnp.zeros((NP - N, D_H), jnp.float32)


def _tc_bnrelu_body(agg_ref, degp_ref, b_ref, g_ref, bt_ref, h_ref):
  dinv = _dinv_from_degp(degp_ref[...])
  z = agg_ref[0:N, :] * dinv + b_ref[...]
  mu = jnp.mean(z, axis=0, keepdims=True)
  var = jnp.mean((z - mu) ** 2, axis=0, keepdims=True)
  h = (z - mu) * lax.rsqrt(var + 1e-5) * g_ref[...] + bt_ref[...]
  h_ref[...] = jnp.maximum(h, 0.0)


def _tc_matmul_body(split_out, h_ref, degp_ref, w_ref, tab_ref):
  dinv = _dinv_from_degp(degp_ref[...])
  hn = jnp.dot(h_ref[...], w_ref[...], preferred_element_type=jnp.float32)
  hn = hn * dinv
  if split_out:
    tab_ref[0:N, :] = hn
    tab_ref[N:NP, :] = jnp.zeros((NP - N, D_H), jnp.float32)
  else:
    tab_ref[0:N, 0:D_OUT] = hn
    tab_ref[0:N, D_OUT:D3P] = jnp.zeros((N, D3P - D_OUT), jnp.float32)
    tab_ref[N:NP, :] = jnp.zeros((NP - N, D3P), jnp.float32)


def _tc_final_body(agg3_ref, degp_ref, b3_ref, out_ref):
  dinv = _dinv_from_degp(degp_ref[...])
  z = agg3_ref[0:N, 0:D_OUT] * dinv + b3_ref[...]
  m = jnp.max(z, axis=1, keepdims=True)
  lse = jnp.log(jnp.sum(jnp.exp(z - m), axis=1, keepdims=True)) + m
  out_ref[...] = z - lse


def _tc_pre(x, W1, degp):
  return pl.pallas_call(
      _tc_pre_body,
      out_shape=jax.ShapeDtypeStruct((NP, D_H), jnp.float32),
  )(x, W1, degp)


def _tc_mid(agg, degp, b, gamma, beta, W, split_out):
  h = pl.pallas_call(
      _tc_bnrelu_body,
      out_shape=jax.ShapeDtypeStruct((N, D_H), jnp.float32),
  )(agg, degp, b, gamma, beta)
  out_shape = jax.ShapeDtypeStruct(
      (NP, D_H if split_out else D3P), jnp.float32)
  return pl.pallas_call(
      functools.partial(_tc_matmul_body, split_out),
      out_shape=out_shape,
  )(h, degp, W)


def _tc_final(agg3, degp, b3):
  return pl.pallas_call(
      _tc_final_body,
      out_shape=jax.ShapeDtypeStruct((N, D_OUT), jnp.float32),
  )(agg3, degp, b3)


def kernel(x, edge_index, W1, b1, W2, b2, W3, b3, gamma, beta):
  src = edge_index[0]
  dst = edge_index[1]
  pad_e = EP - E
  src_p = jnp.concatenate([src, jnp.zeros((pad_e,), src.dtype)])
  trash = N + (jnp.arange(pad_e, dtype=dst.dtype) % 8)
  dst_p = jnp.concatenate([dst, trash])
  src_r = src_p.reshape(NROWSP, CHUNK)
  dst_r = dst_p.reshape(NROWSP, CHUNK)
  ones16 = jnp.concatenate(
      [jnp.ones((CHUNK, 16), jnp.float32),
       jnp.zeros((RPSP, 16), jnp.float32)], axis=0)
  b1r = b1.reshape(1, D_H)
  b2r = b2.reshape(1, D_H)
  b3r = b3.reshape(1, D_OUT)
  gr = gamma.reshape(1, D_H)
  btr = beta.reshape(1, D_H)

  degp = _sc_degree(dst_r, ones16)
  tab1 = _tc_pre(x, W1, degp)
  agg1 = _sc_aggregate(tab1, src_r, dst_r, NQ)
  tab2 = _tc_mid(agg1, degp, b1r, gr, btr, W2, split_out=True)
  agg2 = _sc_aggregate(tab2, src_r, dst_r, NQ)
  tab3 = _tc_mid(agg2, degp, b2r, gr, btr, W3, split_out=False)
  agg3 = _sc_aggregate(tab3, src_r, dst_r, NQ3)
  return _tc_final(agg3, degp, b3r)
